# baseline (speedup 1.0000x reference)
import functools

import jax
import jax.numpy as jnp
from jax import lax
from jax.experimental import pallas as pl
from jax.experimental.pallas import tpu as pltpu
from jax.experimental.pallas import tpu_sc as plsc

_NC = 2
_NS = 16
_NW = _NC * _NS
_CHUNK = 128
_NSLOT = 5


def _sc_gather(W2, idx, n):
    b_per_w = n // _NW
    n_chunks = b_per_w // _CHUNK
    assert n_chunks % _NSLOT == 0
    mesh = plsc.VectorSubcoreMesh(core_axis_name="c", subcore_axis_name="s")

    @functools.partial(
        pl.kernel,
        mesh=mesh,
        out_type=jax.ShapeDtypeStruct((n, 128), W2.dtype),
        scratch_types=[
            pltpu.VMEM((b_per_w,), jnp.int32),
            pltpu.VMEM((_NSLOT, _CHUNK, 128), W2.dtype),
            pltpu.SemaphoreType.DMA((_NSLOT,)),
            pltpu.SemaphoreType.DMA((_NSLOT,)),
        ],
    )
    def gather_kernel(w_hbm, idx_hbm, out_hbm, idx_v, rows_v, gsem, wsem):
        wid = lax.axis_index("s") * _NC + lax.axis_index("c")
        base = wid * b_per_w
        pltpu.sync_copy(idx_hbm.at[pl.ds(base, b_per_w)], idx_v)

        def gather_desc(i, slot):
            return pltpu.make_async_copy(
                w_hbm.at[idx_v.at[pl.ds(i * _CHUNK, _CHUNK)]],
                rows_v.at[slot],
                gsem.at[slot],
            )

        def write_desc(i, slot):
            return pltpu.make_async_copy(
                rows_v.at[slot],
                out_hbm.at[pl.ds(base + i * _CHUNK, _CHUNK)],
                wsem.at[slot],
            )

        for s in range(_NSLOT):
            gather_desc(s, s).start()

        @pl.loop(0, n_chunks // _NSLOT)
        def _(r):
            i = r * _NSLOT
            for s in range(_NSLOT):
                gather_desc(i + s, s).wait()
                write_desc(i + s, s).start()

            @pl.when(i + _NSLOT < n_chunks)
            def _():
                for s in range(_NSLOT):
                    write_desc(i + s, s).wait()
                    gather_desc(i + _NSLOT + s, s).start()

        for s in range(_NSLOT):
            write_desc(n_chunks - _NSLOT + s, s).wait()

    return gather_kernel(W2, idx)


def kernel(x, W):
    B, L = x.shape
    V, D = W.shape
    n = B * L
    idx = x.reshape(n)
    W2 = jnp.pad(W, ((0, 0), (0, D)))
    rows = _sc_gather(W2, idx, n)
    return rows[:, :D].reshape(B, L, D)
